# speedup vs baseline: 2.4942x; 2.4942x over previous
import functools

import jax
import jax.numpy as jnp
from jax import lax
from jax.experimental import pallas as pl
from jax.experimental.pallas import tpu as pltpu
from jax.experimental.pallas import tpu_sc as plsc

B = 16384
L = 50
E = 32
V = 1_000_000

PACK = 8
RP = V // PACK
KP = PACK * E
BLK = 5000


def _dot_body(x_ref, wm_ref, o_ref):
    o_ref[...] = jnp.dot(x_ref[...], wm_ref[...],
                         preferred_element_type=jnp.float32)


def _table_dot_w(emb_table, w_row):
    k = jnp.arange(KP)
    wtile = jnp.tile(w_row, PACK)
    wm = wtile[:, None] * ((k[:, None] // E) ==
                           jnp.arange(PACK)[None, :]).astype(jnp.float32)
    packed = emb_table.reshape(RP, KP)
    s8 = pl.pallas_call(
        _dot_body,
        grid=(RP // BLK,),
        in_specs=[
            pl.BlockSpec((BLK, KP), lambda i: (i, 0)),
            pl.BlockSpec((KP, PACK), lambda i: (0, 0)),
        ],
        out_specs=pl.BlockSpec((BLK, PACK), lambda i: (i, 0)),
        out_shape=jax.ShapeDtypeStruct((RP, PACK), jnp.float32),
    )(packed, wm)
    return s8.reshape(V)


NC, NS = 2, 16
NW = NC * NS
NB_PER_TILE = B // NW
NIDX = NB_PER_TILE * L
CHUNK = 128
NCHUNK = NIDX // CHUNK
GROUP = 8
LPAD = 64

_mesh = plsc.VectorSubcoreMesh(core_axis_name="c", subcore_axis_name="s")


@functools.partial(
    pl.kernel,
    mesh=_mesh,
    out_type=jax.ShapeDtypeStruct((NW, LPAD), jnp.float32),
    scratch_types=[
        pltpu.VMEM((NIDX,), jnp.int32),
        pltpu.VMEM((NIDX + LPAD,), jnp.float32),
        pltpu.VMEM((LPAD,), jnp.float32),
        pltpu.SemaphoreType.DMA,
    ],
)
def _sc_segment_sum(inp_hbm, s_hbm, out_hbm, idx_v, vals_v, acc_v, sem):
    cid = lax.axis_index("c")
    sid = lax.axis_index("s")
    wid = sid * NC + cid
    base = wid * NIDX

    pltpu.sync_copy(inp_hbm.at[pl.ds(base, NIDX)], idx_v)

    def gather_group(g, carry):
        for j in range(GROUP):
            off = (g * GROUP + j) * CHUNK
            pltpu.async_copy(s_hbm.at[idx_v.at[pl.ds(off, CHUNK)]],
                             vals_v.at[pl.ds(off, CHUNK)], sem)
        for j in range(GROUP):
            off = (g * GROUP + j) * CHUNK
            pltpu.make_async_copy(s_hbm.at[idx_v.at[pl.ds(off, CHUNK)]],
                                  vals_v.at[pl.ds(off, CHUNK)], sem).wait()
        return carry

    lax.fori_loop(0, NCHUNK // GROUP, gather_group, 0)

    zero = jnp.zeros((16,), jnp.float32)

    def accum(r, accs):
        o = r * L
        return tuple(a + vals_v[pl.ds(o + 16 * q, 16)]
                     for q, a in enumerate(accs))

    accs = lax.fori_loop(0, NB_PER_TILE, accum, (zero,) * 4)
    for q in range(4):
        acc_v[pl.ds(16 * q, 16)] = accs[q]
    pltpu.sync_copy(acc_v, out_hbm.at[wid])


def kernel(inp, emb_table, W, b):
    s = _table_dot_w(emb_table, W.reshape(-1))
    inp_flat = inp.reshape(-1).astype(jnp.int32)
    partials = _sc_segment_sum(inp_flat, s)
    total = partials[:, :L].sum(axis=0)
    return jax.nn.sigmoid(total * (1.0 / B) + b[0])


# speedup vs baseline: 3.0440x; 1.2205x over previous
import functools

import jax
import jax.numpy as jnp
from jax import lax
from jax.experimental import pallas as pl
from jax.experimental.pallas import tpu as pltpu
from jax.experimental.pallas import tpu_sc as plsc

B = 16384
L = 50
E = 32
V = 1_000_000

BLK = 8000


def _dot_body(w_ref, x_ref, o_ref):
    i = pl.program_id(0)
    prod = lax.dot_general(w_ref[...], x_ref[...],
                           (((1,), (1,)), ((), ())),
                           preferred_element_type=jnp.float32)
    o_ref[pl.ds(pl.multiple_of(i * BLK, 128), BLK)] = prod.reshape(BLK)


def _table_dot_w(emb_table, w_row):
    return pl.pallas_call(
        _dot_body,
        grid=(V // BLK,),
        in_specs=[
            pl.BlockSpec((1, E), lambda i: (0, 0)),
            pl.BlockSpec((BLK, E), lambda i: (i, 0)),
        ],
        out_specs=pl.BlockSpec((V,), lambda i: (0,)),
        out_shape=jax.ShapeDtypeStruct((V,), jnp.float32),
    )(w_row.reshape(1, E), emb_table)


NC, NS = 2, 16
NW = NC * NS
NB_PER_TILE = B // NW
NIDX = NB_PER_TILE * L
CHUNK = 128
NCHUNK = NIDX // CHUNK
GROUP = 8
LPAD = 64

_mesh = plsc.VectorSubcoreMesh(core_axis_name="c", subcore_axis_name="s")


@functools.partial(
    pl.kernel,
    mesh=_mesh,
    out_type=jax.ShapeDtypeStruct((NW, LPAD), jnp.float32),
    scratch_types=[
        pltpu.VMEM((NIDX,), jnp.int32),
        pltpu.VMEM((NIDX + LPAD,), jnp.float32),
        pltpu.VMEM((LPAD,), jnp.float32),
        pltpu.SemaphoreType.DMA,
    ],
)
def _sc_segment_sum(inp_hbm, s_hbm, out_hbm, idx_v, vals_v, acc_v, sem):
    cid = lax.axis_index("c")
    sid = lax.axis_index("s")
    wid = sid * NC + cid
    base = wid * NIDX

    pltpu.sync_copy(inp_hbm.at[pl.ds(base, NIDX)], idx_v)

    def gather_group(g, carry):
        for j in range(GROUP):
            off = (g * GROUP + j) * CHUNK
            pltpu.async_copy(s_hbm.at[idx_v.at[pl.ds(off, CHUNK)]],
                             vals_v.at[pl.ds(off, CHUNK)], sem)
        for j in range(GROUP):
            off = (g * GROUP + j) * CHUNK
            pltpu.make_async_copy(s_hbm.at[idx_v.at[pl.ds(off, CHUNK)]],
                                  vals_v.at[pl.ds(off, CHUNK)], sem).wait()
        return carry

    lax.fori_loop(0, NCHUNK // GROUP, gather_group, 0)

    zero = jnp.zeros((16,), jnp.float32)

    def accum(r, accs):
        o = r * L
        return tuple(a + vals_v[pl.ds(o + 16 * q, 16)]
                     for q, a in enumerate(accs))

    accs = lax.fori_loop(0, NB_PER_TILE, accum, (zero,) * 4)
    for q in range(4):
        acc_v[pl.ds(16 * q, 16)] = accs[q]
    pltpu.sync_copy(acc_v, out_hbm.at[wid])


def kernel(inp, emb_table, W, b):
    s = _table_dot_w(emb_table, W.reshape(-1).astype(jnp.float32))
    inp_flat = inp.reshape(-1).astype(jnp.int32)
    partials = _sc_segment_sum(inp_flat, s)
    total = partials[:, :L].sum(axis=0)
    return jax.nn.sigmoid(total * (1.0 / B) + b[0])
